# Initial kernel scaffold; baseline (speedup 1.0000x reference)
import jax
import jax.numpy as jnp
from jax.experimental import pallas as pl


def kernel(z, edge_index, W_in, b_in, W_msg, b_msg, W_upd, b_upd, W_out, b_out):
    raise NotImplementedError("write your pallas kernel here")


# speedup vs baseline: 3.5662x; 3.5662x over previous
import functools

import jax
import jax.numpy as jnp
from jax import lax
from jax.experimental import pallas as pl
from jax.experimental.pallas import tpu as pltpu
from jax.experimental.pallas import tpu_sc as plsc

N = 10000
H = 128
NC, NS = 2, 16
NW = NC * NS
K = 128
ROW_T = 632
N_PAD = NS * ROW_T
BM = 1000


def _sc_edge_pass(message, src, dst, zeros_tile):
    E_pad = src.shape[0]
    C = E_pad // (NW * K)
    mesh = plsc.VectorSubcoreMesh(
        core_axis_name="c", subcore_axis_name="s", num_cores=NC, num_subcores=NS
    )

    @functools.partial(
        pl.kernel,
        out_type=jax.ShapeDtypeStruct((NC, N_PAD, H), jnp.float32),
        mesh=mesh,
        scratch_types=[
            pltpu.VMEM((K,), jnp.int32),
            pltpu.VMEM((K,), jnp.int32),
            pltpu.VMEM((K, H), jnp.float32),
            pltpu.VMEM_SHARED((N_PAD, H), jnp.float32),
            pltpu.SemaphoreType.DMA,
        ],
    )
    def k(msg_hbm, src_hbm, dst_hbm, zeros_hbm, out_hbm, idx_s, idx_d, rows, agg, sem):
        c = lax.axis_index("c")
        s = lax.axis_index("s")
        wid = c * NS + s
        r0 = s * ROW_T
        pltpu.sync_copy(zeros_hbm, agg.at[pl.ds(r0, ROW_T)])
        plsc.subcore_barrier()
        base0 = wid * C * K

        def body(j, carry):
            base = base0 + j * K
            pltpu.sync_copy(src_hbm.at[pl.ds(base, K)], idx_s)
            pltpu.async_copy(msg_hbm.at[idx_s], rows, sem).wait()
            pltpu.sync_copy(dst_hbm.at[pl.ds(base, K)], idx_d)
            pltpu.sync_copy(rows, agg.at[idx_d], add=True)
            return carry

        lax.fori_loop(0, C, body, 0)
        plsc.subcore_barrier()
        pltpu.sync_copy(agg.at[pl.ds(r0, ROW_T)], out_hbm.at[c, pl.ds(r0, ROW_T)])

    return k(message, src, dst, zeros_tile)


def _row_specs(n):
    return [pl.BlockSpec((BM, H), lambda i: (i, 0)) for _ in range(n)]


def _w_specs(n):
    return [pl.BlockSpec((H, H), lambda i: (0, 0)), pl.BlockSpec((1, H), lambda i: (0, 0))] * n


def _mm(x, w, b):
    return jnp.dot(x, w, preferred_element_type=jnp.float32) + b


def _tc_in_body(z_ref, wi_ref, bi_ref, wm_ref, bm_ref, state_ref, msg_ref):
    s = jnp.maximum(_mm(z_ref[...], wi_ref[...], bi_ref[...]), 0.0)
    state_ref[...] = s
    msg_ref[...] = jnp.maximum(_mm(s, wm_ref[...], bm_ref[...]), 0.0)


def _tc_upd_body(state_ref, p_ref, wu_ref, bu_ref, wm_ref, bm_ref, state_out, msg_ref):
    agg = p_ref[0] + p_ref[1]
    s = state_ref[...] + jnp.maximum(_mm(agg, wu_ref[...], bu_ref[...]), 0.0)
    state_out[...] = s
    msg_ref[...] = jnp.maximum(_mm(s, wm_ref[...], bm_ref[...]), 0.0)


def _tc_fin_body(state_ref, p_ref, wu_ref, bu_ref, wo_ref, bo_ref, out_ref):
    agg = p_ref[0] + p_ref[1]
    s = state_ref[...] + jnp.maximum(_mm(agg, wu_ref[...], bu_ref[...]), 0.0)
    out_ref[...] = _mm(s, wo_ref[...], bo_ref[...])


_PART_SPEC = pl.BlockSpec((NC, BM, H), lambda i: (0, i, 0))
_GRID = (N // BM,)
_ROW_SHAPE = jax.ShapeDtypeStruct((N, H), jnp.float32)


def kernel(z, edge_index, W_in, b_in, W_msg, b_msg, W_upd, b_upd, W_out, b_out):
    src = edge_index[0]
    dst = edge_index[1]
    E = src.shape[0]
    E_pad = ((E + NW * K - 1) // (NW * K)) * (NW * K)
    pad = E_pad - E
    if pad:
        src = jnp.concatenate([src, jnp.zeros((pad,), jnp.int32)])
        dst = jnp.concatenate([dst, jnp.full((pad,), N, jnp.int32)])
    zeros_tile = jnp.zeros((ROW_T, H), jnp.float32)

    b_in2 = b_in.reshape(1, H)
    rounds = W_msg.shape[0]

    state, msg = pl.pallas_call(
        _tc_in_body,
        grid=_GRID,
        in_specs=_row_specs(1) + _w_specs(2),
        out_specs=_row_specs(2),
        out_shape=(_ROW_SHAPE, _ROW_SHAPE),
    )(z, W_in, b_in2, W_msg[0], b_msg[0].reshape(1, H))

    for r in range(rounds - 1):
        partials = _sc_edge_pass(msg, src, dst, zeros_tile)
        state, msg = pl.pallas_call(
            _tc_upd_body,
            grid=_GRID,
            in_specs=_row_specs(1) + [_PART_SPEC] + _w_specs(2),
            out_specs=_row_specs(2),
            out_shape=(_ROW_SHAPE, _ROW_SHAPE),
        )(state, partials, W_upd[r], b_upd[r].reshape(1, H),
          W_msg[r + 1], b_msg[r + 1].reshape(1, H))

    partials = _sc_edge_pass(msg, src, dst, zeros_tile)
    W_out_pad = jnp.zeros((H, H), jnp.float32).at[:, : W_out.shape[1]].set(W_out)
    b_out_pad = jnp.zeros((1, H), jnp.float32).at[0, : b_out.shape[0]].set(b_out)
    out = pl.pallas_call(
        _tc_fin_body,
        grid=_GRID,
        in_specs=_row_specs(1) + [_PART_SPEC] + _w_specs(2),
        out_specs=_row_specs(1)[0],
        out_shape=_ROW_SHAPE,
    )(state, partials, W_upd[rounds - 1], b_upd[rounds - 1].reshape(1, H),
      W_out_pad, b_out_pad)
    return out[:, : W_out.shape[1]]
